# Initial kernel scaffold; baseline (speedup 1.0000x reference)
import functools

import jax
import jax.numpy as jnp
from jax import lax
from jax.experimental import pallas as pl
from jax.experimental.pallas import tpu as pltpu
from jax.experimental.pallas import tpu_sc as plsc

_NC = 2
_NS = 16
_NW = _NC * _NS
_L = 16
_CHUNK = 64


def _body(nseq, rows_per_w, d, x_hbm, cub_hbm, tok_hbm, w_hbm, out_hbm,
          cu_vm, tok_vm, idx_vm, x_vm, w_vm, sem):
    cid = lax.axis_index("c")
    sid = lax.axis_index("s")
    wid = sid * _NC + cid
    base = wid * rows_per_w

    pltpu.sync_copy(cub_hbm, cu_vm)
    bounds = [cu_vm[j] for j in range(1, nseq + 1)]

    for k in range(rows_per_w // _CHUNK):
        start = base + k * _CHUNK
        pltpu.sync_copy(x_hbm.at[pl.ds(start, _CHUNK)], x_vm)
        for g in range(_CHUNK // _L):
            t = start + g * _L + lax.iota(jnp.int32, _L)
            seg = jnp.zeros((_L,), jnp.int32)
            for b in bounds:
                seg = seg + (b <= t).astype(jnp.int32)
            idx_vm[pl.ds(g * _L, _L)] = t + seg + 1
        pltpu.async_copy(x_vm, out_hbm.at[idx_vm], sem).wait()

    @pl.when(wid == 0)
    def _():
        pltpu.sync_copy(tok_hbm, tok_vm)
        for i in range(_L):
            pltpu.sync_copy(w_hbm, w_vm.at[pl.ds(i, 1)])
        pltpu.async_copy(w_vm, out_hbm.at[tok_vm], sem).wait()


def kernel(x_flat, cu_seqlens, weight):
    t_tok, d = x_flat.shape
    nseq = cu_seqlens.shape[0] - 1
    assert t_tok % (_NW * _CHUNK) == 0
    rows_per_w = t_tok // _NW

    cu = cu_seqlens.astype(jnp.int32)
    cu_b = jnp.broadcast_to(cu[:, None], (nseq + 1, _L))
    token_pos = cu[:-1] + jnp.arange(nseq, dtype=jnp.int32)
    tok16 = jnp.tile(token_pos, _L // nseq)

    mesh = plsc.VectorSubcoreMesh(core_axis_name="c", subcore_axis_name="s")
    run = pl.kernel(
        functools.partial(_body, nseq, rows_per_w, d),
        out_type=jax.ShapeDtypeStruct((t_tok + nseq, d), x_flat.dtype),
        mesh=mesh,
        scratch_types=[
            pltpu.VMEM((nseq + 1, _L), jnp.int32),
            pltpu.VMEM((_L,), jnp.int32),
            pltpu.VMEM((_CHUNK,), jnp.int32),
            pltpu.VMEM((_CHUNK, d), jnp.float32),
            pltpu.VMEM((_L, d), jnp.float32),
            pltpu.SemaphoreType.DMA,
        ],
    )
    return run(x_flat, cu_b, tok16, weight)


# speedup vs baseline: 1.7379x; 1.7379x over previous
import functools

import jax
import jax.numpy as jnp
from jax import lax
from jax.experimental import pallas as pl
from jax.experimental.pallas import tpu as pltpu
from jax.experimental.pallas import tpu_sc as plsc

_NC = 2
_NS = 16
_NW = _NC * _NS
_L = 16
_CHUNK = 64


def _body(nseq, rows_per_w, d, x_hbm, cub_hbm, tok_hbm, w_hbm, out_hbm,
          cu_vm, tok_vm, idx_vm, x_vm, w_vm, sem):
    cid = lax.axis_index("c")
    sid = lax.axis_index("s")
    wid = sid * _NC + cid
    base = wid * rows_per_w

    pltpu.sync_copy(cub_hbm, cu_vm)
    bounds = [cu_vm[j] for j in range(1, nseq + 1)]

    for k in range(rows_per_w // _CHUNK):
        start = base + k * _CHUNK
        pltpu.sync_copy(x_hbm.at[pl.ds(start, _CHUNK)], x_vm)
        for g in range(_CHUNK // _L):
            t = start + g * _L + lax.iota(jnp.int32, _L)
            pos = t + 1
            for b in bounds:
                pos = jnp.where(b <= t, pos + 1, pos)
            idx_vm[pl.ds(g * _L, _L)] = pos
        pltpu.async_copy(x_vm, out_hbm.at[idx_vm], sem).wait()

    @pl.when(wid == 0)
    def _():
        pltpu.sync_copy(tok_hbm, tok_vm)
        for i in range(_L):
            pltpu.sync_copy(w_hbm, w_vm.at[pl.ds(i, 1)])
        pltpu.async_copy(w_vm, out_hbm.at[tok_vm], sem).wait()


def kernel(x_flat, cu_seqlens, weight):
    t_tok, d = x_flat.shape
    nseq = cu_seqlens.shape[0] - 1
    assert t_tok % (_NW * _CHUNK) == 0
    rows_per_w = t_tok // _NW

    cu = cu_seqlens.astype(jnp.int32)
    cu_b = jnp.broadcast_to(cu[:, None], (nseq + 1, _L))
    token_pos = cu[:-1] + jnp.arange(nseq, dtype=jnp.int32)
    tok16 = jnp.tile(token_pos, _L // nseq)

    mesh = plsc.VectorSubcoreMesh(core_axis_name="c", subcore_axis_name="s")
    run = pl.kernel(
        functools.partial(_body, nseq, rows_per_w, d),
        out_type=jax.ShapeDtypeStruct((t_tok + nseq, d), x_flat.dtype),
        mesh=mesh,
        scratch_types=[
            pltpu.VMEM((nseq + 1, _L), jnp.int32),
            pltpu.VMEM((_L,), jnp.int32),
            pltpu.VMEM((_CHUNK,), jnp.int32),
            pltpu.VMEM((_CHUNK, d), jnp.float32),
            pltpu.VMEM((_L, d), jnp.float32),
            pltpu.SemaphoreType.DMA,
        ],
    )
    return run(x_flat, cu_b, tok16, weight)


# speedup vs baseline: 2.1832x; 1.2562x over previous
import functools

import jax
import jax.numpy as jnp
from jax import lax
from jax.experimental import pallas as pl
from jax.experimental.pallas import tpu as pltpu
from jax.experimental.pallas import tpu_sc as plsc

_NC = 2
_NS = 16
_NW = _NC * _NS
_L = 16
_CHUNK = 32
_NBUF = 3


def _body(nseq, rows_per_w, d, x_hbm, cub_hbm, tok_hbm, w_hbm, out_hbm,
          cu_vm, tok_vm, idx_vm, x_vm, w_vm, in_sems, out_sems, sem_w):
    cid = lax.axis_index("c")
    sid = lax.axis_index("s")
    wid = sid * _NC + cid
    base = wid * rows_per_w
    nchunks = rows_per_w // _CHUNK

    def w_stage_copies():
        return [pltpu.make_async_copy(tok_hbm, tok_vm, sem_w)] + [
            pltpu.make_async_copy(w_hbm, w_vm.at[pl.ds(i, 1)], sem_w)
            for i in range(_L)
        ]

    @pl.when(wid == 0)
    def _():
        for cp in w_stage_copies():
            cp.start()

    pltpu.sync_copy(cub_hbm, cu_vm)
    bounds = [cu_vm[j] for j in range(1, nseq + 1)]

    def start_in(k):
        b = k % _NBUF
        return pltpu.async_copy(
            x_hbm.at[pl.ds(base + k * _CHUNK, _CHUNK)], x_vm[b], in_sems[b])

    ins = {k: start_in(k) for k in range(min(_NBUF, nchunks))}
    for k in range(nchunks):
        b = k % _NBUF
        start = base + k * _CHUNK
        for g in range(_CHUNK // _L):
            t = start + g * _L + lax.iota(jnp.int32, _L)
            pos = t + 1
            for bound in bounds:
                pos = jnp.where(bound <= t, pos + 1, pos)
            idx_vm[b][pl.ds(g * _L, _L)] = pos
        ins[k].wait()
        out_cp = pltpu.async_copy(x_vm[b], out_hbm.at[idx_vm[b]], out_sems[b])
        if k + _NBUF < nchunks:
            out_cp.wait()
            ins[k + _NBUF] = start_in(k + _NBUF)
        else:
            out_cp.wait()

    @pl.when(wid == 0)
    def _():
        for cp in w_stage_copies():
            cp.wait()
        pltpu.async_copy(w_vm, out_hbm.at[tok_vm], sem_w).wait()


def kernel(x_flat, cu_seqlens, weight):
    t_tok, d = x_flat.shape
    nseq = cu_seqlens.shape[0] - 1
    assert t_tok % (_NW * _CHUNK) == 0
    rows_per_w = t_tok // _NW

    cu = cu_seqlens.astype(jnp.int32)
    cu_b = jnp.broadcast_to(cu[:, None], (nseq + 1, _L))
    token_pos = cu[:-1] + jnp.arange(nseq, dtype=jnp.int32)
    tok16 = jnp.tile(token_pos, _L // nseq)

    mesh = plsc.VectorSubcoreMesh(core_axis_name="c", subcore_axis_name="s")
    run = pl.kernel(
        functools.partial(_body, nseq, rows_per_w, d),
        out_type=jax.ShapeDtypeStruct((t_tok + nseq, d), x_flat.dtype),
        mesh=mesh,
        scratch_types=[
            pltpu.VMEM((nseq + 1, _L), jnp.int32),
            pltpu.VMEM((_L,), jnp.int32),
            [pltpu.VMEM((_CHUNK,), jnp.int32) for _ in range(_NBUF)],
            [pltpu.VMEM((_CHUNK, d), jnp.float32) for _ in range(_NBUF)],
            pltpu.VMEM((_L, d), jnp.float32),
            [pltpu.SemaphoreType.DMA for _ in range(_NBUF)],
            [pltpu.SemaphoreType.DMA for _ in range(_NBUF)],
            pltpu.SemaphoreType.DMA,
        ],
    )
    return run(x_flat, cu_b, tok16, weight)
